# baseline (speedup 1.0000x reference)
import jax
import jax.numpy as jnp
from jax import lax
from jax.experimental import pallas as pl
from jax.experimental.pallas import tpu as pltpu
from jax.experimental.pallas import tpu_sc as plsc

N = 10000
E = 320000
D = 128
H = 256
C = 32
G = 512

NC, NS, L = 2, 16, 16
NW = NC * NS

_SC_PARAMS = pltpu.CompilerParams(needs_layout_passes=False)


def _sc_mesh():
    return plsc.VectorSubcoreMesh(core_axis_name="c", subcore_axis_name="s",
                                  num_cores=NC, num_subcores=NS)


def _mm_relu_body(x_ref, w_ref, b_ref, o_ref):
    acc = jnp.dot(x_ref[...], w_ref[...], preferred_element_type=jnp.float32)
    o_ref[...] = jnp.maximum(acc + b_ref[...], 0.0)


def _mm_relu(x, W, b, block_rows=1000):
    n, d = x.shape
    h = W.shape[1]
    return pl.pallas_call(
        _mm_relu_body,
        grid=(n // block_rows,),
        in_specs=[
            pl.BlockSpec((block_rows, d), lambda i: (i, 0)),
            pl.BlockSpec((d, h), lambda i: (0, 0)),
            pl.BlockSpec((1, h), lambda i: (0, 0)),
        ],
        out_specs=pl.BlockSpec((block_rows, h), lambda i: (i, 0)),
        out_shape=jax.ShapeDtypeStruct((n, h), jnp.float32),
    )(x, W, b.reshape(1, h))


def _heads_body(ap_ref, mp_ref, ct_ref, wm_ref, wx_ref, wa_ref,
                bm_ref, bx_ref, ba_ref, o_ref):
    ap = ap_ref[...]
    mx = mp_ref[...]
    ct = ct_ref[...]
    mean = ap / jnp.maximum(ct, 1.0)
    acc = jnp.dot(mean, wm_ref[...], preferred_element_type=jnp.float32)
    acc = acc + jnp.dot(mx, wx_ref[...], preferred_element_type=jnp.float32)
    acc = acc + jnp.dot(ap, wa_ref[...], preferred_element_type=jnp.float32)
    o_ref[...] = (acc + bm_ref[...] + bx_ref[...] + ba_ref[...]) / 3.0


def _heads(ap, mp, ct, Wm, bm, Wx, bx, Wa, ba):
    return pl.pallas_call(
        _heads_body,
        out_shape=jax.ShapeDtypeStruct((G, C), jnp.float32),
    )(ap, mp, ct, Wm, Wx, Wa,
      bm.reshape(1, C), bx.reshape(1, C), ba.reshape(1, C))


RPW = 320
DUMMY = RPW
CH = 2000
NCHUNK = E // CH
NSTEP = CH // L
GB = 32


def _edge_body(h_hbm, src_hbm, dst_hbm, agg_hbm,
               acc, sbuf, dbuf, csrc, cdst, gbuf, sem):
    c = lax.axis_index("c")
    s = lax.axis_index("s")
    wid = c * NS + s
    lo = wid * RPW
    lanes = lax.iota(jnp.int32, L)
    zeros_f = jnp.zeros((L,), jnp.float32)
    zeros_i = jnp.zeros((L,), jnp.int32)

    def zrow(r, carry):
        for q in range(H // L):
            acc[r, pl.ds(q * L, L)] = zeros_f
        return carry

    lax.fori_loop(0, RPW + 1, zrow, 0)

    def chunk(ch, carry):
        base = pl.multiple_of(ch * CH, CH)
        pltpu.sync_copy(src_hbm.at[pl.ds(base, CH)], sbuf)
        pltpu.sync_copy(dst_hbm.at[pl.ds(base, CH)], dbuf)

        def scan(i, off):
            d = dbuf[pl.ds(i * L, L)]
            sv = sbuf[pl.ds(i * L, L)]
            dl = d - lo
            m = (dl >= 0) & (dl < RPW)
            plsc.store_compressed(csrc.at[pl.ds(off, L)], sv, mask=m)
            plsc.store_compressed(cdst.at[pl.ds(off, L)], dl, mask=m)
            return off + jnp.sum(m.astype(jnp.int32))

        off = lax.fori_loop(0, NSTEP, scan, jnp.int32(0))

        csrc[pl.ds(off, L)] = zeros_i
        csrc[pl.ds(off + L, L)] = zeros_i
        cdst[pl.ds(off, L)] = zeros_i + DUMMY
        cdst[pl.ds(off + L, L)] = zeros_i + DUMMY
        nb = (off + GB - 1) // GB

        def gbatch(b, carry):
            pltpu.async_copy(h_hbm.at[csrc.at[pl.ds(b * GB, GB)]], gbuf,
                             sem).wait()
            for g in range(GB // L):
                cv = cdst[pl.ds(b * GB + g * L, L)]
                for j in range(L):
                    rj = jnp.sum(jnp.where(lanes == j, cv, 0))
                    for q in range(H // L):
                        cur = acc[rj, pl.ds(q * L, L)]
                        acc[rj, pl.ds(q * L, L)] = (
                            cur + gbuf[g * L + j, pl.ds(q * L, L)])
            return carry

        lax.fori_loop(0, nb, gbatch, 0)
        return carry

    lax.fori_loop(0, NCHUNK, chunk, 0)

    lo_r = pl.multiple_of(lo, RPW)

    @pl.when(wid < NW - 1)
    def _():
        pltpu.sync_copy(acc.at[pl.ds(0, RPW)],
                        agg_hbm.at[pl.ds(lo_r, RPW)])

    @pl.when(wid == NW - 1)
    def _():
        rem = N - (NW - 1) * RPW
        pltpu.sync_copy(acc.at[pl.ds(0, rem)],
                        agg_hbm.at[pl.ds(lo_r, rem)])


def _edge_agg(h, src, dst):
    f = pl.kernel(
        _edge_body,
        out_type=jax.ShapeDtypeStruct((N, H), jnp.float32),
        mesh=_sc_mesh(),
        compiler_params=_SC_PARAMS,
        scratch_types=[
            pltpu.VMEM((RPW + 1, H), jnp.float32),
            pltpu.VMEM((CH,), jnp.int32),
            pltpu.VMEM((CH,), jnp.int32),
            pltpu.VMEM((CH + 2 * GB,), jnp.int32),
            pltpu.VMEM((CH + 2 * GB,), jnp.int32),
            pltpu.VMEM((GB, H), jnp.float32),
            pltpu.SemaphoreType.DMA,
        ],
    )
    return f(h, src, dst)


GPW = G // NW
BS_ITERS = 14


def _pool_body(h2_hbm, batch_hbm, ap_hbm, mp_hbm, ct_hbm,
               batch_v, rbuf, grow, cnt_v, sem):
    c = lax.axis_index("c")
    s = lax.axis_index("s")
    wid = c * NS + s
    g0 = wid * GPW
    lanes = lax.iota(jnp.int32, L)

    pltpu.sync_copy(batch_hbm, batch_v)

    def bsearch(targets):
        def step(_, lohi):
            lo_, hi_ = lohi
            active = lo_ < hi_
            mid = (lo_ + hi_) // 2
            v = plsc.load_gather(batch_v, [jnp.minimum(mid, N - 1)])
            lt = (v < targets) & active
            ge = jnp.logical_not(v < targets) & active
            return (jnp.where(lt, mid + 1, lo_), jnp.where(ge, mid, hi_))

        lo_, _ = lax.fori_loop(0, BS_ITERS, step,
                               (jnp.zeros((L,), jnp.int32),
                                jnp.full((L,), N, jnp.int32)))
        return lo_

    starts = bsearch(g0 + lanes)
    ends = bsearch(g0 + 1 + lanes)
    cnt_v[pl.ds(0, GPW)] = (ends - starts).astype(jnp.float32)
    pltpu.sync_copy(cnt_v, ct_hbm.at[pl.ds(pl.multiple_of(g0, GPW), GPW)])

    nq = H // L

    def group(t, carry):
        sg = jnp.sum(jnp.where(lanes == t, starts, 0))
        eg = jnp.sum(jnp.where(lanes == t, ends, 0))
        nb = (eg - sg + (L - 1)) // L

        def gbatch(b, sm):
            sums, maxs = sm
            rb = sg + b * L
            r0 = jnp.minimum(rb, N - L)
            r0f = pl.multiple_of(r0 * H, H)
            pltpu.sync_copy(h2_hbm.at[pl.ds(r0f, L * H)], rbuf)
            sums = list(sums)
            maxs = list(maxs)
            for j in range(L):
                r = r0 + j
                fv = ((r >= rb) & (r < eg)).astype(jnp.float32)
                for q in range(nq):
                    v = rbuf[pl.ds(j * H + q * L, L)] * fv
                    sums[q] = sums[q] + v
                    maxs[q] = jnp.maximum(maxs[q], v)
            return (tuple(sums), tuple(maxs))

        z = tuple(jnp.zeros((L,), jnp.float32) for _ in range(nq))
        sums, maxs = lax.fori_loop(0, nb, gbatch, (z, z))
        for q in range(nq):
            grow[pl.ds(q * L, L)] = sums[q]
            grow[pl.ds(H + q * L, L)] = maxs[q]
        gf = pl.multiple_of((g0 + t) * H, H)
        pltpu.sync_copy(grow.at[pl.ds(0, H)], ap_hbm.at[pl.ds(gf, H)])
        pltpu.sync_copy(grow.at[pl.ds(H, H)], mp_hbm.at[pl.ds(gf, H)])
        return carry

    lax.fori_loop(0, GPW, group, 0)


def _pool(h2, batch):
    f = pl.kernel(
        _pool_body,
        out_type=(jax.ShapeDtypeStruct((G * H,), jnp.float32),
                  jax.ShapeDtypeStruct((G * H,), jnp.float32),
                  jax.ShapeDtypeStruct((G,), jnp.float32)),
        mesh=_sc_mesh(),
        compiler_params=_SC_PARAMS,
        scratch_types=[
            pltpu.VMEM((N,), jnp.int32),
            pltpu.VMEM((L * H,), jnp.float32),
            pltpu.VMEM((2 * H,), jnp.float32),
            pltpu.VMEM((GPW,), jnp.float32),
            pltpu.SemaphoreType.DMA,
        ],
    )
    ap, mp, ct = f(h2.reshape(N * H), batch)
    return ap.reshape(G, H), mp.reshape(G, H), ct


def kernel(x, edge_index, batch, W1, b1, W2, b2, Wm, bm, Wx, bx, Wa, ba):
    src = edge_index[0]
    dst = edge_index[1]
    h = _mm_relu(x, W1, b1)
    agg = _edge_agg(h, src, dst)
    h2 = _mm_relu(agg, W2, b2)
    ap, mp, ct = _pool(h2, batch)
    return _heads(ap, mp, ct.reshape(G, 1), Wm, bm, Wx, bx, Wa, ba)


# speedup vs baseline: 1.2178x; 1.2178x over previous
import jax
import jax.numpy as jnp
from jax import lax
from jax.experimental import pallas as pl
from jax.experimental.pallas import tpu as pltpu
from jax.experimental.pallas import tpu_sc as plsc

N = 10000
E = 320000
D = 128
H = 256
C = 32
G = 512

NC, NS, L = 2, 16, 16
NW = NC * NS

_SC_PARAMS = pltpu.CompilerParams(needs_layout_passes=False)


def _sc_mesh():
    return plsc.VectorSubcoreMesh(core_axis_name="c", subcore_axis_name="s",
                                  num_cores=NC, num_subcores=NS)


def _mm_relu_body(x_ref, w_ref, b_ref, o_ref):
    acc = jnp.dot(x_ref[...], w_ref[...], preferred_element_type=jnp.float32)
    o_ref[...] = jnp.maximum(acc + b_ref[...], 0.0)


def _mm_relu(x, W, b, block_rows=1000):
    n, d = x.shape
    h = W.shape[1]
    return pl.pallas_call(
        _mm_relu_body,
        grid=(n // block_rows,),
        in_specs=[
            pl.BlockSpec((block_rows, d), lambda i: (i, 0)),
            pl.BlockSpec((d, h), lambda i: (0, 0)),
            pl.BlockSpec((1, h), lambda i: (0, 0)),
        ],
        out_specs=pl.BlockSpec((block_rows, h), lambda i: (i, 0)),
        out_shape=jax.ShapeDtypeStruct((n, h), jnp.float32),
    )(x, W, b.reshape(1, h))


def _heads_body(ap_ref, mp_ref, ct_ref, wm_ref, wx_ref, wa_ref,
                bm_ref, bx_ref, ba_ref, o_ref):
    ap = ap_ref[...]
    mx = mp_ref[...]
    ct = ct_ref[...]
    mean = ap / jnp.maximum(ct, 1.0)
    acc = jnp.dot(mean, wm_ref[...], preferred_element_type=jnp.float32)
    acc = acc + jnp.dot(mx, wx_ref[...], preferred_element_type=jnp.float32)
    acc = acc + jnp.dot(ap, wa_ref[...], preferred_element_type=jnp.float32)
    o_ref[...] = (acc + bm_ref[...] + bx_ref[...] + ba_ref[...]) / 3.0


def _heads(ap, mp, ct, Wm, bm, Wx, bx, Wa, ba):
    return pl.pallas_call(
        _heads_body,
        out_shape=jax.ShapeDtypeStruct((G, C), jnp.float32),
    )(ap, mp, ct, Wm, Wx, Wa,
      bm.reshape(1, C), bx.reshape(1, C), ba.reshape(1, C))


RPW = 320
DUMMY = RPW
ACC_W = (RPW + 1) * H
CH = 6400
NCHUNK = E // CH
NSTEP = CH // L
GB = 32
CPAD = CH + 2 * GB


def _edge_body(h_hbm, src_hbm, dst_hbm, agg_hbm,
               acc, sbuf, dbuf, csrc, cdst, gbuf0, gbuf1, sem0, sem1):
    c = lax.axis_index("c")
    s = lax.axis_index("s")
    wid = c * NS + s
    lo = wid * RPW
    lanes = lax.iota(jnp.int32, L)
    zeros_f = jnp.zeros((L,), jnp.float32)
    zeros_i = jnp.zeros((L,), jnp.int32)

    def zrow(j, carry):
        acc[pl.ds(j * L, L)] = zeros_f
        return carry

    lax.fori_loop(0, ACC_W // L, zrow, 0)

    def zsrc(j, carry):
        csrc[pl.ds(j * L, L)] = zeros_i
        return carry

    lax.fori_loop(0, CPAD // L, zsrc, 0)

    def chunk(ch, carry):
        base = pl.multiple_of(ch * CH, CH)
        pltpu.sync_copy(src_hbm.at[pl.ds(base, CH)], sbuf)
        pltpu.sync_copy(dst_hbm.at[pl.ds(base, CH)], dbuf)

        def scan(i, off):
            d = dbuf[pl.ds(i * L, L)]
            sv = sbuf[pl.ds(i * L, L)]
            dl = d - lo
            m = (dl >= 0) & (dl < RPW)
            plsc.store_compressed(csrc.at[pl.ds(off, L)], sv, mask=m)
            plsc.store_compressed(cdst.at[pl.ds(off, L)], dl, mask=m)
            return off + jnp.sum(m.astype(jnp.int32))

        off = lax.fori_loop(0, NSTEP, scan, jnp.int32(0))

        csrc[pl.ds(off, L)] = zeros_i
        csrc[pl.ds(off + L, L)] = zeros_i
        cdst[pl.ds(off, L)] = zeros_i + DUMMY
        cdst[pl.ds(off + L, L)] = zeros_i + DUMMY
        nb = (off + GB - 1) // GB

        def accum(buf, b):
            for j in range(GB):
                rj = plsc.load_gather(cdst, [zeros_i + (b * GB + j)])
                rbase = rj * H + lanes
                for q in range(H // L):
                    val = buf[j, pl.ds(q * L, L)]
                    plsc.addupdate_scatter(acc, [rbase + q * L], val)

        def gpair(p, carry):
            b0 = 2 * p
            b1 = 2 * p + 1
            h0 = pltpu.async_copy(h_hbm.at[csrc.at[pl.ds(b0 * GB, GB)]],
                                  gbuf0, sem0)
            h1 = pltpu.async_copy(h_hbm.at[csrc.at[pl.ds(b1 * GB, GB)]],
                                  gbuf1, sem1)
            h0.wait()
            accum(gbuf0, b0)
            h1.wait()

            @pl.when(b1 < nb)
            def _():
                accum(gbuf1, b1)

            return carry

        lax.fori_loop(0, (nb + 1) // 2, gpair, 0)
        return carry

    lax.fori_loop(0, NCHUNK, chunk, 0)

    lo_flat = pl.multiple_of(lo * H, H)

    @pl.when(wid < NW - 1)
    def _():
        pltpu.sync_copy(acc.at[pl.ds(0, RPW * H)],
                        agg_hbm.at[pl.ds(lo_flat, RPW * H)])

    @pl.when(wid == NW - 1)
    def _():
        rem = N - (NW - 1) * RPW
        pltpu.sync_copy(acc.at[pl.ds(0, rem * H)],
                        agg_hbm.at[pl.ds(lo_flat, rem * H)])


def _edge_agg(h, src, dst):
    f = pl.kernel(
        _edge_body,
        out_type=jax.ShapeDtypeStruct((N * H,), jnp.float32),
        mesh=_sc_mesh(),
        compiler_params=_SC_PARAMS,
        scratch_types=[
            pltpu.VMEM((ACC_W,), jnp.float32),
            pltpu.VMEM((CH,), jnp.int32),
            pltpu.VMEM((CH,), jnp.int32),
            pltpu.VMEM((CPAD,), jnp.int32),
            pltpu.VMEM((CPAD,), jnp.int32),
            pltpu.VMEM((GB, H), jnp.float32),
            pltpu.VMEM((GB, H), jnp.float32),
            pltpu.SemaphoreType.DMA,
            pltpu.SemaphoreType.DMA,
        ],
    )
    return f(h, src, dst).reshape(N, H)


GPW = G // NW
BS_ITERS = 14


def _pool_body(h2_hbm, batch_hbm, ap_hbm, mp_hbm, ct_hbm,
               batch_v, rbuf, grow, cnt_v, sem):
    c = lax.axis_index("c")
    s = lax.axis_index("s")
    wid = c * NS + s
    g0 = wid * GPW
    lanes = lax.iota(jnp.int32, L)

    pltpu.sync_copy(batch_hbm, batch_v)

    def bsearch(targets):
        def step(_, lohi):
            lo_, hi_ = lohi
            active = lo_ < hi_
            mid = (lo_ + hi_) // 2
            v = plsc.load_gather(batch_v, [jnp.minimum(mid, N - 1)])
            lt = (v < targets) & active
            ge = jnp.logical_not(v < targets) & active
            return (jnp.where(lt, mid + 1, lo_), jnp.where(ge, mid, hi_))

        lo_, _ = lax.fori_loop(0, BS_ITERS, step,
                               (jnp.zeros((L,), jnp.int32),
                                jnp.full((L,), N, jnp.int32)))
        return lo_

    starts = bsearch(g0 + lanes)
    ends = bsearch(g0 + 1 + lanes)
    cnt_v[pl.ds(0, GPW)] = (ends - starts).astype(jnp.float32)
    pltpu.sync_copy(cnt_v, ct_hbm.at[pl.ds(pl.multiple_of(g0, GPW), GPW)])

    nq = H // L

    def group(t, carry):
        sg = jnp.sum(jnp.where(lanes == t, starts, 0))
        eg = jnp.sum(jnp.where(lanes == t, ends, 0))
        nb = (eg - sg + (L - 1)) // L

        def gbatch(b, sm):
            sums, maxs = sm
            rb = sg + b * L
            r0 = jnp.minimum(rb, N - L)
            r0f = pl.multiple_of(r0 * H, H)
            pltpu.sync_copy(h2_hbm.at[pl.ds(r0f, L * H)], rbuf)
            sums = list(sums)
            maxs = list(maxs)
            for j in range(L):
                r = r0 + j
                fv = ((r >= rb) & (r < eg)).astype(jnp.float32)
                for q in range(nq):
                    v = rbuf[pl.ds(j * H + q * L, L)] * fv
                    sums[q] = sums[q] + v
                    maxs[q] = jnp.maximum(maxs[q], v)
            return (tuple(sums), tuple(maxs))

        z = tuple(jnp.zeros((L,), jnp.float32) for _ in range(nq))
        sums, maxs = lax.fori_loop(0, nb, gbatch, (z, z))
        for q in range(nq):
            grow[pl.ds(q * L, L)] = sums[q]
            grow[pl.ds(H + q * L, L)] = maxs[q]
        gf = pl.multiple_of((g0 + t) * H, H)
        pltpu.sync_copy(grow.at[pl.ds(0, H)], ap_hbm.at[pl.ds(gf, H)])
        pltpu.sync_copy(grow.at[pl.ds(H, H)], mp_hbm.at[pl.ds(gf, H)])
        return carry

    lax.fori_loop(0, GPW, group, 0)


def _pool(h2, batch):
    f = pl.kernel(
        _pool_body,
        out_type=(jax.ShapeDtypeStruct((G * H,), jnp.float32),
                  jax.ShapeDtypeStruct((G * H,), jnp.float32),
                  jax.ShapeDtypeStruct((G,), jnp.float32)),
        mesh=_sc_mesh(),
        compiler_params=_SC_PARAMS,
        scratch_types=[
            pltpu.VMEM((N,), jnp.int32),
            pltpu.VMEM((L * H,), jnp.float32),
            pltpu.VMEM((2 * H,), jnp.float32),
            pltpu.VMEM((GPW,), jnp.float32),
            pltpu.SemaphoreType.DMA,
        ],
    )
    ap, mp, ct = f(h2.reshape(N * H), batch)
    return ap.reshape(G, H), mp.reshape(G, H), ct


def kernel(x, edge_index, batch, W1, b1, W2, b2, Wm, bm, Wx, bx, Wa, ba):
    src = edge_index[0]
    dst = edge_index[1]
    h = _mm_relu(x, W1, b1)
    agg = _edge_agg(h, src, dst)
    h2 = _mm_relu(agg, W2, b2)
    ap, mp, ct = _pool(h2, batch)
    return _heads(ap, mp, ct.reshape(G, 1), Wm, bm, Wx, bx, Wa, ba)
